# baseline (speedup 1.0000x reference)
import functools

import jax
import jax.numpy as jnp
from jax import lax
from jax.experimental import pallas as pl
from jax.experimental.pallas import tpu as pltpu
from jax.experimental.pallas import tpu_sc as plsc

_D_MODEL = 2048
_N_EXP = 16
_BLK = 1024
_NEG = float(jnp.finfo(jnp.float32).min)

_NC = 2
_NS = 16
_NW = _NC * _NS
_L = 16
_CHUNKS = 4


def _matmul_body(x_ref, w_ref, l_ref):
    xb = x_ref[...]
    w = w_ref[...]
    l_ref[...] = jax.lax.dot_general(
        w, xb, (((1,), (1,)), ((), ())),
        preferred_element_type=jnp.float32)


def _tc_logits(x, W):
    tokens = x.shape[0]
    return pl.pallas_call(
        _matmul_body,
        grid=(tokens // _BLK,),
        in_specs=[
            pl.BlockSpec((_BLK, _D_MODEL), lambda i: (i, 0)),
            pl.BlockSpec((_N_EXP, _D_MODEL), lambda i: (0, 0)),
        ],
        out_specs=pl.BlockSpec((_N_EXP, _BLK), lambda i: (0, i)),
        out_shape=jax.ShapeDtypeStruct((_N_EXP, tokens), jnp.float32),
        compiler_params=pltpu.CompilerParams(
            dimension_semantics=("arbitrary",),
        ),
    )(x, W)


def _sc_body(tpw, l_hbm, g_hbm, i_hbm, lv, g1v, g2v, i1v, i2v):
    wid = lax.axis_index("s") * _NC + lax.axis_index("c")
    base = wid * tpw
    pltpu.sync_copy(l_hbm.at[:, pl.ds(base, tpw)], lv)
    for g in range(tpw // _L):
        off = g * _L
        v = [lv[e, pl.ds(off, _L)] for e in range(_N_EXP)]
        m1 = v[0]
        for e in range(1, _N_EXP):
            m1 = jnp.maximum(m1, v[e])
        i1 = jnp.zeros((_L,), jnp.int32)
        for e in range(_N_EXP - 1, -1, -1):
            i1 = jnp.where(v[e] == m1, jnp.int32(e), i1)
        wv = [jnp.where(i1 == jnp.int32(e), _NEG, v[e]) for e in range(_N_EXP)]
        m2 = wv[0]
        for e in range(1, _N_EXP):
            m2 = jnp.maximum(m2, wv[e])
        i2 = jnp.zeros((_L,), jnp.int32)
        for e in range(_N_EXP - 1, -1, -1):
            i2 = jnp.where(wv[e] == m2, jnp.int32(e), i2)
        ex = jnp.exp(m2 - m1)
        g1v[pl.ds(off, _L)] = 1.0 / (1.0 + ex)
        g2v[pl.ds(off, _L)] = ex / (1.0 + ex)
        i1v[pl.ds(off, _L)] = i1
        i2v[pl.ds(off, _L)] = i2
    pltpu.sync_copy(g1v, g_hbm.at[0, pl.ds(base, tpw)])
    pltpu.sync_copy(g2v, g_hbm.at[1, pl.ds(base, tpw)])
    pltpu.sync_copy(i1v, i_hbm.at[0, pl.ds(base, tpw)])
    pltpu.sync_copy(i2v, i_hbm.at[1, pl.ds(base, tpw)])


def _sc_top2(logits_t):
    tokens = logits_t.shape[1]
    tpw = tokens // _NW
    mesh = plsc.VectorSubcoreMesh(core_axis_name="c", subcore_axis_name="s")
    f = pl.kernel(
        functools.partial(_sc_body, tpw), mesh=mesh,
        out_type=[
            jax.ShapeDtypeStruct((2, tokens), jnp.float32),
            jax.ShapeDtypeStruct((2, tokens), jnp.int32),
        ],
        scratch_types=[
            pltpu.VMEM((_N_EXP, tpw), jnp.float32),
            pltpu.VMEM((tpw,), jnp.float32),
            pltpu.VMEM((tpw,), jnp.float32),
            pltpu.VMEM((tpw,), jnp.int32),
            pltpu.VMEM((tpw,), jnp.int32),
        ],
    )
    return f(logits_t)


def kernel(x, W):
    tokens = x.shape[0]
    step = tokens // _CHUNKS
    gs, is_ = [], []
    for c in range(_CHUNKS):
        logits_t = _tc_logits(jax.lax.slice_in_dim(x, c * step, (c + 1) * step), W)
        g_t, i_t = _sc_top2(logits_t)
        gs.append(g_t)
        is_.append(i_t)
    gates_t = jnp.concatenate(gs, axis=1)
    indices_t = jnp.concatenate(is_, axis=1)
    return (gates_t.T, indices_t.T)


# speedup vs baseline: 2.6621x; 2.6621x over previous
import jax
import jax.numpy as jnp
from jax.experimental import pallas as pl
from jax.experimental.pallas import tpu as pltpu

_D_MODEL = 2048
_N_EXP = 16
_BLK = 1024
_NEG = float(jnp.finfo(jnp.float32).min)


def _router_body(x_ref, w_ref, g_ref, i_ref):
    xb = x_ref[...]
    w = w_ref[...]
    logits = jax.lax.dot_general(
        w, xb, (((1,), (1,)), ((), ())),
        preferred_element_type=jnp.float32)
    row = jax.lax.broadcasted_iota(jnp.int32, logits.shape, 0)

    m1 = jnp.max(logits, axis=0, keepdims=True)
    i1 = jnp.min(jnp.where(logits == m1, row, _N_EXP), axis=0, keepdims=True)
    masked = jnp.where(row == i1, _NEG, logits)
    m2 = jnp.max(masked, axis=0, keepdims=True)
    i2 = jnp.min(jnp.where(masked == m2, row, _N_EXP), axis=0, keepdims=True)

    e = jnp.exp(m2 - m1)
    g1 = 1.0 / (1.0 + e)
    g2 = e / (1.0 + e)
    g_pair = jnp.concatenate([g1, g2], axis=0)
    i_pair = jnp.concatenate([i1, i2], axis=0)
    g_ref[...] = jax.lax.transpose(g_pair, (1, 0))
    i_ref[...] = jax.lax.transpose(i_pair, (1, 0))


def kernel(x, W):
    tokens = x.shape[0]
    grid = (tokens // _BLK,)
    gates, indices = pl.pallas_call(
        _router_body,
        grid=grid,
        in_specs=[
            pl.BlockSpec((_BLK, _D_MODEL), lambda i: (i, 0)),
            pl.BlockSpec((_N_EXP, _D_MODEL), lambda i: (0, 0)),
        ],
        out_specs=[
            pl.BlockSpec((_BLK, 2), lambda i: (i, 0)),
            pl.BlockSpec((_BLK, 2), lambda i: (i, 0)),
        ],
        out_shape=[
            jax.ShapeDtypeStruct((tokens, 2), jnp.float32),
            jax.ShapeDtypeStruct((tokens, 2), jnp.int32),
        ],
        compiler_params=pltpu.CompilerParams(
            dimension_semantics=("arbitrary",),
        ),
    )(x, W)
    return (gates, indices)


# speedup vs baseline: 3.7566x; 1.4112x over previous
import jax
import jax.numpy as jnp
from jax.experimental import pallas as pl
from jax.experimental.pallas import tpu as pltpu

_D_MODEL = 2048
_N_EXP = 16
_BLK = 1024
_NEG = float(jnp.finfo(jnp.float32).min)


def _router_body(x_ref, w_ref, g_ref, i_ref):
    xb = x_ref[...]
    w = w_ref[...]
    logits = jax.lax.dot_general(
        w, xb, (((1,), (1,)), ((), ())),
        preferred_element_type=jnp.float32)
    row = jax.lax.broadcasted_iota(jnp.int32, logits.shape, 0)

    m1 = jnp.max(logits, axis=0, keepdims=True)
    i1 = jnp.min(jnp.where(logits == m1, row, _N_EXP), axis=0, keepdims=True)
    masked = jnp.where(row == i1, _NEG, logits)
    m2 = jnp.max(masked, axis=0, keepdims=True)
    i2 = jnp.min(jnp.where(masked == m2, row, _N_EXP), axis=0, keepdims=True)

    e = jnp.exp(m2 - m1)
    g1 = 1.0 / (1.0 + e)
    g2 = e / (1.0 + e)
    g_ref[...] = jnp.concatenate([g1, g2], axis=0)
    i_ref[...] = jnp.concatenate([i1, i2], axis=0)


def kernel(x, W):
    tokens = x.shape[0]
    grid = (tokens // _BLK,)
    gates_t, indices_t = pl.pallas_call(
        _router_body,
        grid=grid,
        in_specs=[
            pl.BlockSpec((_BLK, _D_MODEL), lambda i: (i, 0)),
            pl.BlockSpec((_N_EXP, _D_MODEL), lambda i: (0, 0)),
        ],
        out_specs=[
            pl.BlockSpec((2, _BLK), lambda i: (0, i)),
            pl.BlockSpec((2, _BLK), lambda i: (0, i)),
        ],
        out_shape=[
            jax.ShapeDtypeStruct((2, tokens), jnp.float32),
            jax.ShapeDtypeStruct((2, tokens), jnp.int32),
        ],
        compiler_params=pltpu.CompilerParams(
            dimension_semantics=("parallel",),
        ),
    )(x, W)
    return (gates_t.T, indices_t.T)
